# Initial kernel scaffold; baseline (speedup 1.0000x reference)
import jax
import jax.numpy as jnp
from jax.experimental import pallas as pl


def kernel(x, edge_index, W_fc, b_fc, b_gc1, W_gc2, b_gc2):
    raise NotImplementedError("write your pallas kernel here")


# speedup vs baseline: 4.2831x; 4.2831x over previous
import functools

import jax
import jax.numpy as jnp
from jax import lax
from jax.experimental import pallas as pl
from jax.experimental.pallas import tpu as pltpu
from jax.experimental.pallas import tpu_sc as plsc

N = 10000
E = 320000
D = 128

NC = 2
NS = 16
NW = NC * NS
C = 80
ROWS = E // C
ROWS_PER_TILE = ROWS // NW
ROWS_PER_SUB = ROWS // NS
N_PAD = 10240
NODES_PER_SUB = N_PAD // NS

_MESH = plsc.VectorSubcoreMesh(
    core_axis_name="c", subcore_axis_name="s", num_cores=NC, num_subcores=NS)


HROWS = N_PAD // D


def _sc_degrees_body(eidx, zeros8, deg_out, idx_v, hist, buf, iota80, deg_sh):
    c = lax.axis_index("c")
    s = lax.axis_index("s")
    r = lax.iota(jnp.int32, 16)
    zvec = jnp.zeros((16,), jnp.float32)
    ones_vec = jnp.ones((16,), jnp.float32)

    def zrow(i, carry):
        for j in range(8):
            hist[i, pl.ds(16 * j, 16)] = zvec
        return carry

    lax.fori_loop(0, HROWS, zrow, 0)
    buf[pl.ds(16, 16)] = jnp.full((16,), -2, jnp.int32)
    for k in range(5):
        iota80[pl.ds(16 * k, 16)] = r + 16 * k

    @pl.when(s < HROWS // 8)
    def _():
        pltpu.sync_copy(zeros8, deg_sh.at[pl.ds(8 * s, 8)])

    plsc.subcore_barrier()

    def body(i, carry):
        row = s * ROWS_PER_SUB + i
        pltpu.sync_copy(eidx.at[c, row], idx_v)
        for j in range(C // 16):
            x = idx_v[pl.ds(16 * j, 16)]
            plsc.addupdate_scatter(
                hist,
                [lax.shift_right_logical(x, 7), lax.bitwise_and(x, 127)],
                ones_vec)
        return carry

    lax.fori_loop(0, ROWS_PER_SUB, body, 0)
    pltpu.sync_copy(hist, deg_sh.at[iota80], add=True)
    plsc.subcore_barrier()

    @pl.when(s < HROWS // 8)
    def _():
        pltpu.sync_copy(deg_sh.at[pl.ds(8 * s, 8)],
                        deg_out.at[c, pl.ds(8 * s, 8)])


def _sc_msgpass_body(hs, sidx, didx, zeros, out, sidx_v, didx_v, rows_v, agg_sh, sem):
    c = lax.axis_index("c")
    s = lax.axis_index("s")
    t = c * NS + s
    pltpu.sync_copy(zeros, agg_sh.at[pl.ds(s * NODES_PER_SUB, NODES_PER_SUB)])
    plsc.subcore_barrier()

    def body(i, carry):
        row = t * ROWS_PER_TILE + i
        pltpu.sync_copy(sidx.at[row], sidx_v)
        pltpu.async_copy(hs.at[sidx_v], rows_v, sem).wait()
        pltpu.sync_copy(didx.at[row], didx_v)
        pltpu.sync_copy(rows_v, agg_sh.at[didx_v], add=True)
        return carry

    lax.fori_loop(0, ROWS_PER_TILE, body, 0)
    plsc.subcore_barrier()
    pltpu.sync_copy(agg_sh.at[pl.ds(s * NODES_PER_SUB, NODES_PER_SUB)],
                    out.at[c, pl.ds(s * NODES_PER_SUB, NODES_PER_SUB)])


def _make_sc_degrees(interpret=False):
    return pl.kernel(
        _sc_degrees_body,
        out_type=jax.ShapeDtypeStruct((NC, HROWS, D), jnp.float32),
        mesh=_MESH,
        scratch_types=[
            pltpu.VMEM((C,), jnp.int32),
            pltpu.VMEM((HROWS, D), jnp.float32),
            pltpu.VMEM((32,), jnp.int32),
            pltpu.VMEM((HROWS,), jnp.int32),
            pltpu.VMEM_SHARED((HROWS, D), jnp.float32),
        ],
        compiler_params=pltpu.CompilerParams(needs_layout_passes=False),
        interpret=interpret,
    )


def _make_sc_msgpass(interpret=False):
    return pl.kernel(
        _sc_msgpass_body,
        out_type=jax.ShapeDtypeStruct((NC, N_PAD, D), jnp.float32),
        mesh=_MESH,
        scratch_types=[
            pltpu.VMEM((C,), jnp.int32),
            pltpu.VMEM((C,), jnp.int32),
            pltpu.VMEM((C, D), jnp.float32),
            pltpu.VMEM_SHARED((N_PAD, D), jnp.float32),
            pltpu.SemaphoreType.DMA,
        ],
        interpret=interpret,
    )


_sc_degrees = _make_sc_degrees()
_sc_msgpass = _make_sc_msgpass()


def _norms(deg_ref):
    ds = deg_ref[0].reshape(N_PAD)[:N]
    dd = deg_ref[1].reshape(N_PAD)[:N]
    ns = jnp.where(ds > 0, lax.rsqrt(ds), 0.0)
    nd = jnp.where(dd > 0, lax.rsqrt(dd), 0.0)
    return ns, nd


def _tc1_body(x_ref, w_ref, b_ref, deg_ref, out_ref):
    ns, _ = _norms(deg_ref)
    h = jnp.dot(x_ref[...], w_ref[...], preferred_element_type=jnp.float32)
    out_ref[...] = (h + b_ref[...]) * ns[:, None]


_tc1 = pl.pallas_call(
    _tc1_body, out_shape=jax.ShapeDtypeStruct((N, D), jnp.float32))


def _tc2_body(p_ref, deg_ref, b_ref, out_ref):
    ns, nd = _norms(deg_ref)
    agg = (p_ref[0, :N] + p_ref[1, :N]) * nd[:, None]
    h = jnp.maximum(agg + b_ref[...], 0.0)
    out_ref[...] = h * ns[:, None]


_tc2 = pl.pallas_call(
    _tc2_body, out_shape=jax.ShapeDtypeStruct((N, D), jnp.float32))


def _tc3_body(p_ref, deg_ref, w_ref, b_ref, out_ref):
    _, nd = _norms(deg_ref)
    agg = (p_ref[0, :N] + p_ref[1, :N]) * nd[:, None]
    h = jnp.dot(agg, w_ref[...], preferred_element_type=jnp.float32)
    out_ref[...] = jnp.maximum(h + b_ref[...], 0.0)


_tc3 = pl.pallas_call(
    _tc3_body, out_shape=jax.ShapeDtypeStruct((N, D), jnp.float32))


def kernel(x, edge_index, W_fc, b_fc, b_gc1, W_gc2, b_gc2):
    ei = edge_index.astype(jnp.int32)
    eidx3 = ei.reshape(2, ROWS, C)
    sidx = eidx3[0]
    didx = eidx3[1]
    zeros8 = jnp.zeros((8, D), jnp.float32)
    zerosD = jnp.zeros((NODES_PER_SUB, D), jnp.float32)

    deg = _sc_degrees(eidx3, zeros8)
    hs = _tc1(x, W_fc, b_fc.reshape(1, D), deg)
    p1 = _sc_msgpass(hs, sidx, didx, zerosD)
    hs1 = _tc2(p1, deg, b_gc1.reshape(1, D))
    p2 = _sc_msgpass(hs1, sidx, didx, zerosD)
    return _tc3(p2, deg, W_gc2, b_gc2.reshape(1, D))


# speedup vs baseline: 11.5621x; 2.6995x over previous
import jax
import jax.numpy as jnp
from jax import lax
from jax.experimental import pallas as pl
from jax.experimental.pallas import tpu as pltpu
from jax.experimental.pallas import tpu_sc as plsc

N = 10000
E = 320000
D = 128

NC = 2
NS = 16
NW = NC * NS
C = 80
ROWS = E // C
ROWS_PER_TILE = ROWS // NW
ROWS_PER_SUB = ROWS // NS
N_PAD = 10240
NODES_PER_SUB = N_PAD // NS
HROWS = N_PAD // D

_MESH = plsc.VectorSubcoreMesh(
    core_axis_name="c", subcore_axis_name="s", num_cores=NC, num_subcores=NS)


def _sc_degrees_body(eidx, zeros8, deg_out, idx_v, hist, iota80, deg_sh):
    c = lax.axis_index("c")
    s = lax.axis_index("s")
    r = lax.iota(jnp.int32, 16)
    zvec = jnp.zeros((16,), jnp.float32)
    ones_vec = jnp.ones((16,), jnp.float32)

    def zrow(i, carry):
        for j in range(8):
            hist[i, pl.ds(16 * j, 16)] = zvec
        return carry

    lax.fori_loop(0, HROWS, zrow, 0)
    for k in range(5):
        iota80[pl.ds(16 * k, 16)] = r + 16 * k

    @pl.when(s < HROWS // 8)
    def _():
        pltpu.sync_copy(zeros8, deg_sh.at[pl.ds(8 * s, 8)])

    plsc.subcore_barrier()

    pltpu.sync_copy(eidx.at[c, s], idx_v)

    def body(i, carry):
        for j in range(C // 16):
            x = idx_v[i, pl.ds(16 * j, 16)]
            plsc.addupdate_scatter(
                hist,
                [lax.shift_right_logical(x, 7), lax.bitwise_and(x, 127)],
                ones_vec)
        return carry

    lax.fori_loop(0, ROWS_PER_SUB, body, 0)
    pltpu.sync_copy(hist, deg_sh.at[iota80], add=True)
    plsc.subcore_barrier()

    @pl.when(s < HROWS // 8)
    def _():
        pltpu.sync_copy(deg_sh.at[pl.ds(8 * s, 8)],
                        deg_out.at[c, pl.ds(8 * s, 8)])


def _sc_msgpass_body(hs, sidx, didx, zeros, out,
                     sidx_all, didx_v0, didx_v1, buf0, buf1, agg_sh,
                     sem0, sem1, semd0, semd1, semi):
    c = lax.axis_index("c")
    s = lax.axis_index("s")
    t = c * NS + s
    pltpu.async_copy(sidx.at[t], sidx_all, semi)
    pltpu.sync_copy(zeros, agg_sh.at[pl.ds(s * NODES_PER_SUB, NODES_PER_SUB)])
    pltpu.make_async_copy(sidx.at[t], sidx_all, semi).wait()
    plsc.subcore_barrier()

    def gather(i, buf, sem):
        pltpu.async_copy(hs.at[sidx_all.at[i]], buf, sem)

    def gwait(buf, sem):
        pltpu.make_async_copy(hs.at[sidx_all.at[0]], buf, sem).wait()

    def dload(i, dbuf, sem):
        pltpu.async_copy(didx.at[t, i], dbuf, sem)

    def dwait(dbuf, sem):
        pltpu.make_async_copy(didx.at[t, 0], dbuf, sem).wait()

    def scat(dbuf, buf):
        pltpu.sync_copy(buf, agg_sh.at[dbuf], add=True)

    gather(0, buf0, sem0)
    dload(0, didx_v0, semd0)

    def body(k, carry):
        i = 2 * k
        gather(i + 1, buf1, sem1)
        dload(i + 1, didx_v1, semd1)
        gwait(buf0, sem0)
        dwait(didx_v0, semd0)
        scat(didx_v0, buf0)
        gather(i + 2, buf0, sem0)
        dload(i + 2, didx_v0, semd0)
        gwait(buf1, sem1)
        dwait(didx_v1, semd1)
        scat(didx_v1, buf1)
        return carry

    lax.fori_loop(0, (ROWS_PER_TILE - 1) // 2, body, 0)
    gwait(buf0, sem0)
    dwait(didx_v0, semd0)
    scat(didx_v0, buf0)
    plsc.subcore_barrier()
    pltpu.sync_copy(agg_sh.at[pl.ds(s * NODES_PER_SUB, NODES_PER_SUB)],
                    out.at[c, pl.ds(s * NODES_PER_SUB, NODES_PER_SUB)])


def _make_sc_degrees(interpret=False):
    return pl.kernel(
        _sc_degrees_body,
        out_type=jax.ShapeDtypeStruct((NC, HROWS, D), jnp.float32),
        mesh=_MESH,
        scratch_types=[
            pltpu.VMEM((ROWS_PER_SUB, C), jnp.int32),
            pltpu.VMEM((HROWS, D), jnp.float32),
            pltpu.VMEM((HROWS,), jnp.int32),
            pltpu.VMEM_SHARED((HROWS, D), jnp.float32),
        ],
        compiler_params=pltpu.CompilerParams(needs_layout_passes=False),
        interpret=interpret,
    )


def _make_sc_msgpass(interpret=False):
    return pl.kernel(
        _sc_msgpass_body,
        out_type=jax.ShapeDtypeStruct((NC, N_PAD, D), jnp.float32),
        mesh=_MESH,
        scratch_types=[
            pltpu.VMEM((ROWS_PER_TILE, C), jnp.int32),
            pltpu.VMEM((C,), jnp.int32),
            pltpu.VMEM((C,), jnp.int32),
            pltpu.VMEM((C, D), jnp.float32),
            pltpu.VMEM((C, D), jnp.float32),
            pltpu.VMEM_SHARED((N_PAD, D), jnp.float32),
            pltpu.SemaphoreType.DMA,
            pltpu.SemaphoreType.DMA,
            pltpu.SemaphoreType.DMA,
            pltpu.SemaphoreType.DMA,
            pltpu.SemaphoreType.DMA,
        ],
        interpret=interpret,
    )


_sc_degrees = _make_sc_degrees()
_sc_msgpass = _make_sc_msgpass()


def _norms(deg_ref):
    ds = deg_ref[0].reshape(N_PAD)[:N]
    dd = deg_ref[1].reshape(N_PAD)[:N]
    ns = jnp.where(ds > 0, lax.rsqrt(ds), 0.0)
    nd = jnp.where(dd > 0, lax.rsqrt(dd), 0.0)
    return ns, nd


def _tc1_body(x_ref, w_ref, b_ref, deg_ref, out_ref):
    ns, _ = _norms(deg_ref)
    h = jnp.dot(x_ref[...], w_ref[...], preferred_element_type=jnp.float32)
    out_ref[...] = (h + b_ref[...]) * ns[:, None]


_tc1 = pl.pallas_call(
    _tc1_body, out_shape=jax.ShapeDtypeStruct((N, D), jnp.float32))


def _tc2_body(p_ref, deg_ref, b_ref, out_ref):
    ns, nd = _norms(deg_ref)
    agg = (p_ref[0, :N] + p_ref[1, :N]) * nd[:, None]
    h = jnp.maximum(agg + b_ref[...], 0.0)
    out_ref[...] = h * ns[:, None]


_tc2 = pl.pallas_call(
    _tc2_body, out_shape=jax.ShapeDtypeStruct((N, D), jnp.float32))


def _tc3_body(p_ref, deg_ref, w_ref, b_ref, out_ref):
    _, nd = _norms(deg_ref)
    agg = (p_ref[0, :N] + p_ref[1, :N]) * nd[:, None]
    h = jnp.dot(agg, w_ref[...], preferred_element_type=jnp.float32)
    out_ref[...] = jnp.maximum(h + b_ref[...], 0.0)


_tc3 = pl.pallas_call(
    _tc3_body, out_shape=jax.ShapeDtypeStruct((N, D), jnp.float32))


def kernel(x, edge_index, W_fc, b_fc, b_gc1, W_gc2, b_gc2):
    ei = edge_index.astype(jnp.int32)
    eidx3 = ei.reshape(2, NS, ROWS_PER_SUB, C)
    sidx = ei[0].reshape(NW, ROWS_PER_TILE, C)
    didx = ei[1].reshape(NW, ROWS_PER_TILE, C)
    zeros8 = jnp.zeros((8, D), jnp.float32)
    zerosD = jnp.zeros((NODES_PER_SUB, D), jnp.float32)

    deg = _sc_degrees(eidx3, zeros8)
    hs = _tc1(x, W_fc, b_fc.reshape(1, D), deg)
    p1 = _sc_msgpass(hs, sidx, didx, zerosD)
    hs1 = _tc2(p1, deg, b_gc1.reshape(1, D))
    p2 = _sc_msgpass(hs1, sidx, didx, zerosD)
    return _tc3(p2, deg, W_gc2, b_gc2.reshape(1, D))
